# baseline (speedup 1.0000x reference)
import functools

import jax
from jax import lax
import jax.numpy as jnp
from jax.experimental import pallas as pl
from jax.experimental.pallas import tpu as pltpu
from jax.experimental.pallas import tpu_sc as plsc

N_NODES = 50000
N_EDGES = 800000
EMB_DIM = 64
EDGE_DIM = 16
HIDDEN = 128
N_LAYERS = 4
NUM_EMB = 100
NUM_GRAPHS = 128

NODE_BLK = 2000
EDGE_BLK = 3200

NW = 32
EPW = N_EDGES // NW
GCH = 128
GFULL = EPW // GCH
GREM = EPW - GFULL * GCH

QBOUND = (0, 12504, 25008, 37512, 50000)
STRIPE = 784
ACC_ROWS = 16 * STRIPE
DUMMY = 12520
EPT = N_EDGES // 16
SCH = 64
SFULL = EPT // SCH
SREM = EPT - SFULL * SCH
ZCH = 56


def _emb_kernel(x_ref, emb_ref, out_ref):
    x = x_ref[0, 0]
    onehot = (x[:, None] == lax.broadcasted_iota(jnp.int32, (NODE_BLK, 128), 1)).astype(jnp.float32)
    out_ref[...] = jnp.dot(onehot, emb_ref[...], precision=lax.Precision.HIGHEST)


def _embed(x, emb_pad):
    x3 = x.reshape(N_NODES // NODE_BLK, 1, NODE_BLK)
    return pl.pallas_call(
        _emb_kernel,
        grid=(N_NODES // NODE_BLK,),
        in_specs=[
            pl.BlockSpec((1, 1, NODE_BLK), lambda i: (i, 0, 0)),
            pl.BlockSpec((128, 2 * EMB_DIM), lambda i: (0, 0)),
        ],
        out_specs=pl.BlockSpec((NODE_BLK, 2 * EMB_DIM), lambda i: (i, 0)),
        out_shape=jax.ShapeDtypeStruct((N_NODES, 2 * EMB_DIM), jnp.float32),
    )(x3, emb_pad)


def _gather_body(h_ref, dst_ref, src_ref, gi_ref, gj_ref, idx_v, idx_r, rows_v, rows_r, sem):
    wid = lax.axis_index("s") * 2 + lax.axis_index("c")
    base = wid * EPW

    def run(idx_hbm, out_hbm):
        def body(i, carry):
            off = base + i * GCH
            pltpu.sync_copy(idx_hbm.at[pl.ds(off, GCH)], idx_v)
            pltpu.async_copy(h_ref.at[idx_v], rows_v, sem).wait()
            pltpu.sync_copy(rows_v, out_hbm.at[pl.ds(off, GCH)])
            return carry

        lax.fori_loop(0, GFULL, body, 0)
        off = base + GFULL * GCH
        pltpu.sync_copy(idx_hbm.at[pl.ds(off, GREM)], idx_r)
        pltpu.async_copy(h_ref.at[idx_r], rows_r, sem).wait()
        pltpu.sync_copy(rows_r, out_hbm.at[pl.ds(off, GREM)])

    run(dst_ref, gi_ref)
    run(src_ref, gj_ref)


def _gather(h, dst, src):
    return pl.kernel(
        _gather_body,
        out_type=(jax.ShapeDtypeStruct((N_EDGES, 2 * EMB_DIM), jnp.float32),
                  jax.ShapeDtypeStruct((N_EDGES, 2 * EMB_DIM), jnp.float32)),
        mesh=plsc.VectorSubcoreMesh(core_axis_name="c", subcore_axis_name="s"),
        scratch_types=[
            pltpu.VMEM((GCH,), jnp.int32),
            pltpu.VMEM((GREM,), jnp.int32),
            pltpu.VMEM((GCH, 2 * EMB_DIM), jnp.float32),
            pltpu.VMEM((GREM, 2 * EMB_DIM), jnp.float32),
            pltpu.SemaphoreType.DMA,
        ],
    )(h, dst, src)


def _scatter_body(msg_ref, dst_ref, zero_ref, agg_ref, acc_sh, idx_v, idx_r, rows_v, rows_r, sem):
    c = lax.axis_index("c")
    s = lax.axis_index("s")
    ebase = s * EPT

    for p in range(2):
        q = c * 2 + p
        nodebase = q * 12504
        qsize = jnp.where(q < 3, 12504, 12488)

        pltpu.sync_copy(zero_ref, rows_v)

        def zbody(k, carry):
            pltpu.sync_copy(rows_v.at[pl.ds(0, ZCH)], acc_sh.at[pl.ds(s * STRIPE + k * ZCH, ZCH)])
            return carry

        lax.fori_loop(0, STRIPE // ZCH, zbody, 0)
        plsc.subcore_barrier()

        def relocate(idx_ref, n16):
            for j in range(n16):
                v = idx_ref[pl.ds(j * 16, 16)] - nodebase
                ok = (v >= 0) & (v < qsize)
                idx_ref[pl.ds(j * 16, 16)] = jnp.where(ok, v, DUMMY)

        def body(i, carry):
            off = ebase + i * SCH
            pltpu.sync_copy(dst_ref.at[pl.ds(off, SCH)], idx_v)
            pltpu.sync_copy(msg_ref.at[pl.ds(off, SCH)], rows_v)
            relocate(idx_v, SCH // 16)
            pltpu.sync_copy(rows_v, acc_sh.at[idx_v], add=True)
            return carry

        lax.fori_loop(0, SFULL, body, 0)
        off = ebase + SFULL * SCH
        pltpu.sync_copy(dst_ref.at[pl.ds(off, SREM)], idx_r)
        pltpu.sync_copy(msg_ref.at[pl.ds(off, SREM)], rows_r)
        relocate(idx_r, SREM // 16)
        pltpu.sync_copy(rows_r, acc_sh.at[idx_r], add=True)

        plsc.subcore_barrier()

        def fchunk(r0, n):
            pltpu.sync_copy(acc_sh.at[pl.ds(r0, n)], rows_v.at[pl.ds(0, n)])
            pltpu.sync_copy(rows_v.at[pl.ds(0, n)], agg_ref.at[pl.ds(nodebase + r0, n)])

        @pl.when(s < 15)
        def _flush_main():
            def fbody(k, carry):
                fchunk(s * STRIPE + k * ZCH, ZCH)
                return carry
            lax.fori_loop(0, STRIPE // ZCH, fbody, 0)

        @pl.when(s == 15)
        def _flush_tail():
            def fbody(k, carry):
                fchunk(15 * STRIPE + k * ZCH, ZCH)
                return carry
            lax.fori_loop(0, 13, fbody, 0)

        @pl.when(jnp.logical_and(s == 15, q < 3))
        def _flush_tail_rem():
            fchunk(15 * STRIPE + 13 * ZCH, 16)

        plsc.subcore_barrier()


def _scatter(msg, dst, zero_blk):
    return pl.kernel(
        _scatter_body,
        out_type=jax.ShapeDtypeStruct((N_NODES, 2 * EMB_DIM), jnp.float32),
        mesh=plsc.VectorSubcoreMesh(core_axis_name="c", subcore_axis_name="s"),
        scratch_types=[
            pltpu.VMEM_SHARED((ACC_ROWS, 2 * EMB_DIM), jnp.float32),
            pltpu.VMEM((SCH,), jnp.int32),
            pltpu.VMEM((SREM,), jnp.int32),
            pltpu.VMEM((SCH, 2 * EMB_DIM), jnp.float32),
            pltpu.VMEM((SREM, 2 * EMB_DIM), jnp.float32),
            pltpu.SemaphoreType.DMA,
        ],
    )(msg, dst, zero_blk)


def _msg_kernel(gi_ref, gj_ref, ea_ref, W_ref, b_ref, out_ref):
    z = jnp.concatenate([gi_ref[:, :EMB_DIM], gj_ref[:, :EMB_DIM], ea_ref[...]], axis=1)
    pre = jnp.dot(z, W_ref[...]) + b_ref[...]
    gate = jax.nn.sigmoid(pre[:, :EMB_DIM])
    core = jax.nn.softplus(pre[:, EMB_DIM:])
    m = gate * core
    out_ref[...] = jnp.concatenate([m, jnp.zeros_like(m)], axis=1)


def _msg(gi, gj, ea, W, b):
    return pl.pallas_call(
        _msg_kernel,
        grid=(N_EDGES // EDGE_BLK,),
        in_specs=[
            pl.BlockSpec((EDGE_BLK, 2 * EMB_DIM), lambda i: (i, 0)),
            pl.BlockSpec((EDGE_BLK, 2 * EMB_DIM), lambda i: (i, 0)),
            pl.BlockSpec((EDGE_BLK, EDGE_DIM), lambda i: (i, 0)),
            pl.BlockSpec((2 * EMB_DIM + EDGE_DIM, 2 * EMB_DIM), lambda i: (0, 0)),
            pl.BlockSpec((1, 2 * EMB_DIM), lambda i: (0, 0)),
        ],
        out_specs=pl.BlockSpec((EDGE_BLK, 2 * EMB_DIM), lambda i: (i, 0)),
        out_shape=jax.ShapeDtypeStruct((N_EDGES, 2 * EMB_DIM), jnp.float32),
    )(gi, gj, ea, W, b)


def _upd_kernel(h_ref, agg_ref, out_ref):
    hn = jax.nn.softplus(h_ref[:, :EMB_DIM] + agg_ref[:, :EMB_DIM])
    out_ref[...] = jnp.concatenate([hn, jnp.zeros_like(hn)], axis=1)


def _upd(h, agg):
    return pl.pallas_call(
        _upd_kernel,
        grid=(N_NODES // NODE_BLK,),
        in_specs=[
            pl.BlockSpec((NODE_BLK, 2 * EMB_DIM), lambda i: (i, 0)),
            pl.BlockSpec((NODE_BLK, 2 * EMB_DIM), lambda i: (i, 0)),
        ],
        out_specs=pl.BlockSpec((NODE_BLK, 2 * EMB_DIM), lambda i: (i, 0)),
        out_shape=jax.ShapeDtypeStruct((N_NODES, 2 * EMB_DIM), jnp.float32),
    )(h, agg)


def _pool_mlp_kernel(batch_ref, h_ref, W1_ref, b1_ref, W2_ref, b2_ref, Wo_ref, bo_ref,
                     out_ref, sums_ref, cnts_ref):
    i = pl.program_id(0)

    @pl.when(i == 0)
    def _init():
        sums_ref[...] = jnp.zeros_like(sums_ref)
        cnts_ref[...] = jnp.zeros_like(cnts_ref)

    b = batch_ref[0, 0]
    onehot = (b[:, None] == lax.broadcasted_iota(jnp.int32, (NODE_BLK, NUM_GRAPHS), 1)).astype(jnp.float32)
    sums_ref[...] += jnp.dot(onehot.T, h_ref[:, :EMB_DIM], precision=lax.Precision.HIGHEST)
    cnts_ref[...] += jnp.sum(onehot, axis=0, keepdims=True)

    @pl.when(i == pl.num_programs(0) - 1)
    def _finish():
        g = sums_ref[...] / jnp.maximum(cnts_ref[...], 1.0).T
        g = jax.nn.relu(jnp.dot(g, W1_ref[...]) + b1_ref[...])
        g = jax.nn.relu(jnp.dot(g, W2_ref[...]) + b2_ref[...])
        out_ref[...] = jnp.dot(g, Wo_ref[...]) + bo_ref[...]


def _pool_mlp(batch, h, W1, b1, W2, b2, Wo, bo):
    b3 = batch.reshape(N_NODES // NODE_BLK, 1, NODE_BLK)
    out = pl.pallas_call(
        _pool_mlp_kernel,
        grid=(N_NODES // NODE_BLK,),
        in_specs=[
            pl.BlockSpec((1, 1, NODE_BLK), lambda i: (i, 0, 0)),
            pl.BlockSpec((NODE_BLK, 2 * EMB_DIM), lambda i: (i, 0)),
            pl.BlockSpec((EMB_DIM, HIDDEN), lambda i: (0, 0)),
            pl.BlockSpec((1, HIDDEN), lambda i: (0, 0)),
            pl.BlockSpec((HIDDEN, HIDDEN // 2), lambda i: (0, 0)),
            pl.BlockSpec((1, HIDDEN // 2), lambda i: (0, 0)),
            pl.BlockSpec((HIDDEN // 2, 1), lambda i: (0, 0)),
            pl.BlockSpec((1, 1), lambda i: (0, 0)),
        ],
        out_specs=pl.BlockSpec((NUM_GRAPHS, 1), lambda i: (0, 0)),
        out_shape=jax.ShapeDtypeStruct((NUM_GRAPHS, 1), jnp.float32),
        scratch_shapes=[
            pltpu.VMEM((NUM_GRAPHS, EMB_DIM), jnp.float32),
            pltpu.VMEM((1, NUM_GRAPHS), jnp.float32),
        ],
    )(b3, h, W1, b1.reshape(1, -1), W2, b2.reshape(1, -1), Wo, bo.reshape(1, -1))
    return out.reshape(-1)


def kernel(x, edge_index, edge_attr, batch, emb, Wf, bf, Ws, bs, W1, b1, W2, b2, Wo, bo):
    emb_pad = jnp.zeros((128, 2 * EMB_DIM), jnp.float32).at[:NUM_EMB, :EMB_DIM].set(emb)
    h = _embed(x, emb_pad)
    src = edge_index[0]
    dst = edge_index[1]
    zero_blk = jnp.zeros((SCH, 2 * EMB_DIM), jnp.float32)
    Wcat = jnp.concatenate([Wf, Ws], axis=2)
    bcat = jnp.concatenate([bf, bs], axis=1)
    for l in range(N_LAYERS):
        gi, gj = _gather(h, dst, src)
        msg = _msg(gi, gj, edge_attr, Wcat[l], bcat[l].reshape(1, -1))
        agg = _scatter(msg, dst, zero_blk)
        h = _upd(h, agg)
    return _pool_mlp(batch, h, W1, b1, W2, b2, Wo, bo)


# speedup vs baseline: 1.4203x; 1.4203x over previous
import functools

import jax
from jax import lax
import jax.numpy as jnp
from jax.experimental import pallas as pl
from jax.experimental.pallas import tpu as pltpu
from jax.experimental.pallas import tpu_sc as plsc

N_NODES = 50000
N_EDGES = 800000
EMB_DIM = 64
EDGE_DIM = 16
HIDDEN = 128
N_LAYERS = 4
NUM_EMB = 100
NUM_GRAPHS = 128

NODE_BLK = 2000
EDGE_BLK = 3200

NW = 32
EPW = N_EDGES // NW
GCH = 128
GFULL = EPW // GCH
GREM = EPW - GFULL * GCH

QBOUND = (0, 12504, 25008, 37512, 50000)
STRIPE = 784
ACC_ROWS = 16 * STRIPE
DUMMY = 12520
EPT = N_EDGES // 16
SCH = 128
SFULL = EPT // SCH
SREM = EPT - SFULL * SCH
ZCH = 56


def _emb_kernel(x_ref, emb_ref, out_ref):
    x = x_ref[0, 0]
    onehot = (x[:, None] == lax.broadcasted_iota(jnp.int32, (NODE_BLK, 128), 1)).astype(jnp.float32)
    out_ref[...] = jnp.dot(onehot, emb_ref[...], precision=lax.Precision.HIGHEST)


def _embed(x, emb_pad):
    x3 = x.reshape(N_NODES // NODE_BLK, 1, NODE_BLK)
    return pl.pallas_call(
        _emb_kernel,
        grid=(N_NODES // NODE_BLK,),
        in_specs=[
            pl.BlockSpec((1, 1, NODE_BLK), lambda i: (i, 0, 0)),
            pl.BlockSpec((128, 2 * EMB_DIM), lambda i: (0, 0)),
        ],
        out_specs=pl.BlockSpec((NODE_BLK, 2 * EMB_DIM), lambda i: (i, 0)),
        out_shape=jax.ShapeDtypeStruct((N_NODES, 2 * EMB_DIM), jnp.float32),
    )(x3, emb_pad)


def _gather_body(h_ref, dst_ref, src_ref, gi_ref, gj_ref,
                 idx_a, idx_b, idx_r, rows_a, rows_b, rows_r,
                 sema, semb, semw, semw2, semr):
    wid = lax.axis_index("s") * 2 + lax.axis_index("c")
    base = wid * EPW

    def run(idx_hbm, out_hbm):
        def body(i, carry):
            off = base + i * 2 * GCH
            pltpu.sync_copy(idx_hbm.at[pl.ds(off, GCH)], idx_a)
            pltpu.sync_copy(idx_hbm.at[pl.ds(off + GCH, GCH)], idx_b)
            ca = pltpu.async_copy(h_ref.at[idx_a], rows_a, sema)
            cb = pltpu.async_copy(h_ref.at[idx_b], rows_b, semb)
            ca.wait()
            wa = pltpu.async_copy(rows_a, out_hbm.at[pl.ds(off, GCH)], semw)
            cb.wait()
            wb = pltpu.async_copy(rows_b, out_hbm.at[pl.ds(off + GCH, GCH)], semw2)
            wa.wait()
            wb.wait()
            return carry

        lax.fori_loop(0, EPW // (2 * GCH), body, 0)
        off = base + (EPW // (2 * GCH)) * 2 * GCH
        pltpu.sync_copy(idx_hbm.at[pl.ds(off, GCH)], idx_a)
        pltpu.async_copy(h_ref.at[idx_a], rows_a, sema).wait()
        pltpu.sync_copy(rows_a, out_hbm.at[pl.ds(off, GCH)])
        off = off + GCH
        pltpu.sync_copy(idx_hbm.at[pl.ds(off, GREM)], idx_r)
        pltpu.async_copy(h_ref.at[idx_r], rows_r, semr).wait()
        pltpu.sync_copy(rows_r, out_hbm.at[pl.ds(off, GREM)])

    run(dst_ref, gi_ref)
    run(src_ref, gj_ref)


def _gather(h, dst, src):
    return pl.kernel(
        _gather_body,
        out_type=(jax.ShapeDtypeStruct((N_EDGES, 2 * EMB_DIM), jnp.float32),
                  jax.ShapeDtypeStruct((N_EDGES, 2 * EMB_DIM), jnp.float32)),
        mesh=plsc.VectorSubcoreMesh(core_axis_name="c", subcore_axis_name="s"),
        scratch_types=[
            pltpu.VMEM((GCH,), jnp.int32),
            pltpu.VMEM((GCH,), jnp.int32),
            pltpu.VMEM((GREM,), jnp.int32),
            pltpu.VMEM((GCH, 2 * EMB_DIM), jnp.float32),
            pltpu.VMEM((GCH, 2 * EMB_DIM), jnp.float32),
            pltpu.VMEM((GREM, 2 * EMB_DIM), jnp.float32),
            pltpu.SemaphoreType.DMA,
            pltpu.SemaphoreType.DMA,
            pltpu.SemaphoreType.DMA,
            pltpu.SemaphoreType.DMA,
            pltpu.SemaphoreType.DMA,
        ],
    )(h, dst, src)


def _scatter_body(msg_ref, dst_ref, zero_ref, agg_ref, acc_sh, idx_v, idx_r, rows_v, rows_r, sem, sem2):
    c = lax.axis_index("c")
    s = lax.axis_index("s")
    ebase = s * EPT

    for p in range(2):
        q = c * 2 + p
        nodebase = q * 12504
        qsize = jnp.where(q < 3, 12504, 12488)

        pltpu.sync_copy(zero_ref, rows_v)

        def zbody(k, carry):
            pltpu.sync_copy(rows_v.at[pl.ds(0, ZCH)], acc_sh.at[pl.ds(s * STRIPE + k * ZCH, ZCH)])
            return carry

        lax.fori_loop(0, STRIPE // ZCH, zbody, 0)
        plsc.subcore_barrier()

        def relocate(idx_ref, n16):
            for j in range(n16):
                v = idx_ref[pl.ds(j * 16, 16)] - nodebase
                ok = (v >= 0) & (v < qsize)
                idx_ref[pl.ds(j * 16, 16)] = jnp.where(ok, v, DUMMY)

        def body(i, carry):
            off = ebase + i * SCH
            ca = pltpu.async_copy(dst_ref.at[pl.ds(off, SCH)], idx_v, sem)
            cb = pltpu.async_copy(msg_ref.at[pl.ds(off, SCH)], rows_v, sem2)
            ca.wait()
            relocate(idx_v, SCH // 16)
            cb.wait()
            pltpu.sync_copy(rows_v, acc_sh.at[idx_v], add=True)
            return carry

        lax.fori_loop(0, SFULL, body, 0)
        off = ebase + SFULL * SCH
        pltpu.sync_copy(dst_ref.at[pl.ds(off, SREM)], idx_r)
        pltpu.sync_copy(msg_ref.at[pl.ds(off, SREM)], rows_r)
        relocate(idx_r, SREM // 16)
        pltpu.sync_copy(rows_r, acc_sh.at[idx_r], add=True)

        plsc.subcore_barrier()

        def fchunk(r0, n):
            pltpu.sync_copy(acc_sh.at[pl.ds(r0, n)], rows_v.at[pl.ds(0, n)])
            pltpu.sync_copy(rows_v.at[pl.ds(0, n)], agg_ref.at[pl.ds(nodebase + r0, n)])

        @pl.when(s < 15)
        def _flush_main():
            def fbody(k, carry):
                fchunk(s * STRIPE + k * ZCH, ZCH)
                return carry
            lax.fori_loop(0, STRIPE // ZCH, fbody, 0)

        @pl.when(s == 15)
        def _flush_tail():
            def fbody(k, carry):
                fchunk(15 * STRIPE + k * ZCH, ZCH)
                return carry
            lax.fori_loop(0, 13, fbody, 0)

        @pl.when(jnp.logical_and(s == 15, q < 3))
        def _flush_tail_rem():
            fchunk(15 * STRIPE + 13 * ZCH, 16)

        plsc.subcore_barrier()


def _scatter(msg, dst, zero_blk):
    return pl.kernel(
        _scatter_body,
        out_type=jax.ShapeDtypeStruct((N_NODES, 2 * EMB_DIM), jnp.float32),
        mesh=plsc.VectorSubcoreMesh(core_axis_name="c", subcore_axis_name="s"),
        scratch_types=[
            pltpu.VMEM_SHARED((ACC_ROWS, 2 * EMB_DIM), jnp.float32),
            pltpu.VMEM((SCH,), jnp.int32),
            pltpu.VMEM((SREM,), jnp.int32),
            pltpu.VMEM((SCH, 2 * EMB_DIM), jnp.float32),
            pltpu.VMEM((SREM, 2 * EMB_DIM), jnp.float32),
            pltpu.SemaphoreType.DMA,
            pltpu.SemaphoreType.DMA,
        ],
    )(msg, dst, zero_blk)


def _msg_kernel(gi_ref, gj_ref, ea_ref, W_ref, b_ref, out_ref):
    z = jnp.concatenate([gi_ref[:, :EMB_DIM], gj_ref[:, :EMB_DIM], ea_ref[...]], axis=1)
    pre = jnp.dot(z, W_ref[...]) + b_ref[...]
    gate = jax.nn.sigmoid(pre[:, :EMB_DIM])
    core = jax.nn.softplus(pre[:, EMB_DIM:])
    m = gate * core
    out_ref[...] = jnp.concatenate([m, jnp.zeros_like(m)], axis=1)


def _msg(gi, gj, ea, W, b):
    return pl.pallas_call(
        _msg_kernel,
        grid=(N_EDGES // EDGE_BLK,),
        in_specs=[
            pl.BlockSpec((EDGE_BLK, 2 * EMB_DIM), lambda i: (i, 0)),
            pl.BlockSpec((EDGE_BLK, 2 * EMB_DIM), lambda i: (i, 0)),
            pl.BlockSpec((EDGE_BLK, EDGE_DIM), lambda i: (i, 0)),
            pl.BlockSpec((2 * EMB_DIM + EDGE_DIM, 2 * EMB_DIM), lambda i: (0, 0)),
            pl.BlockSpec((1, 2 * EMB_DIM), lambda i: (0, 0)),
        ],
        out_specs=pl.BlockSpec((EDGE_BLK, 2 * EMB_DIM), lambda i: (i, 0)),
        out_shape=jax.ShapeDtypeStruct((N_EDGES, 2 * EMB_DIM), jnp.float32),
    )(gi, gj, ea, W, b)


def _upd_kernel(h_ref, agg_ref, out_ref):
    hn = jax.nn.softplus(h_ref[:, :EMB_DIM] + agg_ref[:, :EMB_DIM])
    out_ref[...] = jnp.concatenate([hn, jnp.zeros_like(hn)], axis=1)


def _upd(h, agg):
    return pl.pallas_call(
        _upd_kernel,
        grid=(N_NODES // NODE_BLK,),
        in_specs=[
            pl.BlockSpec((NODE_BLK, 2 * EMB_DIM), lambda i: (i, 0)),
            pl.BlockSpec((NODE_BLK, 2 * EMB_DIM), lambda i: (i, 0)),
        ],
        out_specs=pl.BlockSpec((NODE_BLK, 2 * EMB_DIM), lambda i: (i, 0)),
        out_shape=jax.ShapeDtypeStruct((N_NODES, 2 * EMB_DIM), jnp.float32),
    )(h, agg)


def _pool_mlp_kernel(batch_ref, h_ref, W1_ref, b1_ref, W2_ref, b2_ref, Wo_ref, bo_ref,
                     out_ref, sums_ref, cnts_ref):
    i = pl.program_id(0)

    @pl.when(i == 0)
    def _init():
        sums_ref[...] = jnp.zeros_like(sums_ref)
        cnts_ref[...] = jnp.zeros_like(cnts_ref)

    b = batch_ref[0, 0]
    onehot = (b[:, None] == lax.broadcasted_iota(jnp.int32, (NODE_BLK, NUM_GRAPHS), 1)).astype(jnp.float32)
    sums_ref[...] += jnp.dot(onehot.T, h_ref[:, :EMB_DIM], precision=lax.Precision.HIGHEST)
    cnts_ref[...] += jnp.sum(onehot, axis=0, keepdims=True)

    @pl.when(i == pl.num_programs(0) - 1)
    def _finish():
        g = sums_ref[...] / jnp.maximum(cnts_ref[...], 1.0).T
        g = jax.nn.relu(jnp.dot(g, W1_ref[...]) + b1_ref[...])
        g = jax.nn.relu(jnp.dot(g, W2_ref[...]) + b2_ref[...])
        out_ref[...] = jnp.dot(g, Wo_ref[...]) + bo_ref[...]


def _pool_mlp(batch, h, W1, b1, W2, b2, Wo, bo):
    b3 = batch.reshape(N_NODES // NODE_BLK, 1, NODE_BLK)
    out = pl.pallas_call(
        _pool_mlp_kernel,
        grid=(N_NODES // NODE_BLK,),
        in_specs=[
            pl.BlockSpec((1, 1, NODE_BLK), lambda i: (i, 0, 0)),
            pl.BlockSpec((NODE_BLK, 2 * EMB_DIM), lambda i: (i, 0)),
            pl.BlockSpec((EMB_DIM, HIDDEN), lambda i: (0, 0)),
            pl.BlockSpec((1, HIDDEN), lambda i: (0, 0)),
            pl.BlockSpec((HIDDEN, HIDDEN // 2), lambda i: (0, 0)),
            pl.BlockSpec((1, HIDDEN // 2), lambda i: (0, 0)),
            pl.BlockSpec((HIDDEN // 2, 1), lambda i: (0, 0)),
            pl.BlockSpec((1, 1), lambda i: (0, 0)),
        ],
        out_specs=pl.BlockSpec((NUM_GRAPHS, 1), lambda i: (0, 0)),
        out_shape=jax.ShapeDtypeStruct((NUM_GRAPHS, 1), jnp.float32),
        scratch_shapes=[
            pltpu.VMEM((NUM_GRAPHS, EMB_DIM), jnp.float32),
            pltpu.VMEM((1, NUM_GRAPHS), jnp.float32),
        ],
    )(b3, h, W1, b1.reshape(1, -1), W2, b2.reshape(1, -1), Wo, bo.reshape(1, -1))
    return out.reshape(-1)


def kernel(x, edge_index, edge_attr, batch, emb, Wf, bf, Ws, bs, W1, b1, W2, b2, Wo, bo):
    emb_pad = jnp.zeros((128, 2 * EMB_DIM), jnp.float32).at[:NUM_EMB, :EMB_DIM].set(emb)
    h = _embed(x, emb_pad)
    src = edge_index[0]
    dst = edge_index[1]
    zero_blk = jnp.zeros((SCH, 2 * EMB_DIM), jnp.float32)
    Wcat = jnp.concatenate([Wf, Ws], axis=2)
    bcat = jnp.concatenate([bf, bs], axis=1)
    for l in range(N_LAYERS):
        gi, gj = _gather(h, dst, src)
        msg = _msg(gi, gj, edge_attr, Wcat[l], bcat[l].reshape(1, -1))
        agg = _scatter(msg, dst, zero_blk)
        h = _upd(h, agg)
    return _pool_mlp(batch, h, W1, b1, W2, b2, Wo, bo)
